# baseline (speedup 1.0000x reference)
import functools

import jax
import jax.numpy as jnp
from jax import lax
from jax.experimental import pallas as pl
from jax.experimental.pallas import tpu as pltpu
from jax.experimental.pallas import tpu_sc as plsc

N_NODES = 64
N_GROUPS = 100000
BATCH = 16384

NC = 2
NS = 16
NW = NC * NS
LANES = 16

BPW = BATCH // NW
GCH = 128
NG = BPW // GCH
WROW = 128

BM_W = 8192
BM_T = 8192

S_TC = 43008
NBLK = 445
N_SC = NBLK * GCH
TAIL = N_GROUPS - S_TC - N_SC
PREP_ITERS = 14


def _tp_w_kernel(wt_ref, out_ref):
    x = wt_ref[...]
    xt = x.T
    pad = jnp.zeros((BM_W, WROW - N_NODES), jnp.float32)
    out_ref[...] = jnp.concatenate([xt, pad], axis=1)


def _tp_t_kernel(tt_ref, out_ref):
    out_ref[...] = tt_ref[...].T


def _sc_prep_kernel(wt_hbm, hp1_hbm, in_v, out_v, isem, osem):
    sid = lax.axis_index("s") * NC + lax.axis_index("c")
    lane_iota = lax.iota(jnp.int32, LANES)

    def fire_in(kb, slot):
        b = jnp.minimum(kb * NW + sid, NBLK - 1)
        col0 = pl.multiple_of(S_TC + b * GCH, GCH)
        return pltpu.async_copy(wt_hbm.at[:, pl.ds(col0, GCH)],
                                in_v.at[slot], isem)

    def fire_out(kb, slot):
        b = jnp.minimum(kb * NW + sid, NBLK - 1)
        row0 = pl.multiple_of(b * GCH, GCH)
        return pltpu.async_copy(out_v.at[slot, :, pl.ds(0, WROW)],
                                hp1_hbm.at[pl.ds(row0, GCH)], osem)

    def transpose_block(slot):
        ib = in_v.at[slot]
        ob = out_v.at[slot]

        @pl.loop(0, GCH, step=LANES)
        def _cols(q0):
            row_ids = q0 + lane_iota
            for j in range(N_NODES):
                plsc.store_scatter(
                    ob, [row_ids, jnp.full((LANES,), j, jnp.int32)],
                    ib[j, pl.ds(q0, LANES)])

    in_cp = {0: fire_in(0, 0), 1: fire_in(1, 1)}
    out_cp = {}
    for kb in range(PREP_ITERS):
        slot = kb % 2
        in_cp[kb].wait()
        if kb >= 2:
            out_cp[kb - 2].wait()
        transpose_block(slot)
        out_cp[kb] = fire_out(kb, slot)
        if kb + 2 < PREP_ITERS:
            in_cp[kb + 2] = fire_in(kb + 2, slot)
    out_cp[PREP_ITERS - 2].wait()
    out_cp[PREP_ITERS - 1].wait()


def _tp_tail_kernel(wt_ref, out_ref):
    x = wt_ref[...]
    pad = jnp.zeros((TAIL, WROW - N_NODES), jnp.float32)
    out_ref[...] = jnp.concatenate([x.T, pad], axis=1)


def _sc_dot_kernel(t_hbm, c_hbm, w0_hbm, w1_hbm, w2_hbm, out_hbm, c_v,
                   idx0_v, idx1_v, idx2_v, rows_v, t_v, buf_v, out_v,
                   gsem, tsem):
    wid = lax.axis_index("s") * NC + lax.axis_index("c")
    base = pl.multiple_of(wid * BPW, BPW)

    pltpu.sync_copy(c_hbm.at[wid], c_v)

    t_cp = pltpu.async_copy(t_hbm.at[pl.ds(base, BPW)], t_v, tsem)

    @pl.loop(0, GCH, step=LANES)
    def _split(j):
        for g in range(NG):
            cv = c_v[g, pl.ds(j, LANES)]
            idx0_v[g, pl.ds(j, LANES)] = jnp.where(cv < S_TC, cv, -1)
            idx1_v[g, pl.ds(j, LANES)] = jnp.where(
                (cv >= S_TC) & (cv < S_TC + N_SC), cv - S_TC, -1)
            idx2_v[g, pl.ds(j, LANES)] = jnp.where(
                cv >= S_TC + N_SC, cv - (S_TC + N_SC), -1)

    def fire(g):
        rb = rows_v.at[g % 2]
        cp0 = pltpu.async_copy(
            w0_hbm.at[plsc.Indices(idx0_v.at[g], ignored_value=-1)], rb, gsem)
        cp1 = pltpu.async_copy(
            w1_hbm.at[plsc.Indices(idx1_v.at[g], ignored_value=-1)], rb, gsem)
        cp2 = pltpu.async_copy(
            w2_hbm.at[plsc.Indices(idx2_v.at[g], ignored_value=-1)], rb, gsem)
        return (cp0, cp1, cp2)

    gathers = [fire(0), fire(1)]
    t_cp.wait()

    lane_iota = lax.iota(jnp.int32, LANES)
    nchunk = N_NODES // LANES

    for g in range(NG):
        for cp_h in gathers[g]:
            cp_h.wait()
        rb = rows_v.at[g % 2]

        @pl.loop(0, GCH, step=16)
        def _group(r0):
            row0 = g * GCH + r0
            for r in range(16):
                lrow = r0 + r
                grow = row0 + r
                acc = (rb[lrow, pl.ds(0, LANES)]
                       * t_v[grow, pl.ds(0, LANES)])
                for k in range(1, nchunk):
                    acc = acc + (rb[lrow, pl.ds(k * LANES, LANES)]
                                 * t_v[grow, pl.ds(k * LANES, LANES)])
                plsc.store_scatter(
                    buf_v, [lane_iota, jnp.full((LANES,), r, jnp.int32)], acc)
            tot = buf_v[0, :]
            for l in range(1, 16):
                tot = tot + buf_v[l, :]
            out_v[pl.ds(row0, 16)] = tot

        if g + 2 < NG:
            gathers.append(fire(g + 2))

    pltpu.sync_copy(out_v, out_hbm.at[pl.ds(base, BPW)])


@jax.jit
def kernel(t, c, W):
    c2 = c.reshape(NW, NG, GCH).astype(jnp.int32)
    wt = W.T
    mesh = plsc.VectorSubcoreMesh(core_axis_name="c", subcore_axis_name="s")
    cp = pltpu.CompilerParams(needs_layout_passes=False)

    hp1 = functools.partial(
        pl.kernel,
        mesh=mesh,
        compiler_params=cp,
        out_type=jax.ShapeDtypeStruct((N_SC, WROW), jnp.float32),
        scratch_types=[
            pltpu.VMEM((2, N_NODES, GCH), jnp.float32),
            pltpu.VMEM((2, GCH, WROW + 1), jnp.float32),
            pltpu.SemaphoreType.DMA,
            pltpu.SemaphoreType.DMA,
        ],
    )(_sc_prep_kernel)(wt)

    tc_params = pltpu.CompilerParams(dimension_semantics=("parallel",))
    hp0 = pl.pallas_call(
        _tp_w_kernel,
        out_shape=jax.ShapeDtypeStruct((S_TC, WROW), jnp.float32),
        grid=((S_TC + BM_W - 1) // BM_W,),
        in_specs=[pl.BlockSpec((N_NODES, BM_W), lambda i: (0, i))],
        out_specs=pl.BlockSpec((BM_W, WROW), lambda i: (i, 0)),
        compiler_params=tc_params,
    )(wt)

    wt_tail = jax.lax.slice(wt, (0, S_TC + N_SC), (N_NODES, N_GROUPS))
    hp2 = pl.pallas_call(
        _tp_tail_kernel,
        out_shape=jax.ShapeDtypeStruct((TAIL, WROW), jnp.float32),
    )(wt_tail)

    t2 = pl.pallas_call(
        _tp_t_kernel,
        out_shape=jax.ShapeDtypeStruct((BATCH, N_NODES), jnp.float32),
        grid=(BATCH // BM_T,),
        in_specs=[pl.BlockSpec((N_NODES, BM_T), lambda i: (0, i))],
        out_specs=pl.BlockSpec((BM_T, N_NODES), lambda i: (i, 0)),
        compiler_params=tc_params,
    )(t.T)

    run = functools.partial(
        pl.kernel,
        mesh=mesh,
        compiler_params=cp,
        out_type=jax.ShapeDtypeStruct((BATCH,), jnp.float32),
        scratch_types=[
            pltpu.VMEM((NG, GCH), jnp.int32),
            pltpu.VMEM((NG, GCH), jnp.int32),
            pltpu.VMEM((NG, GCH), jnp.int32),
            pltpu.VMEM((NG, GCH), jnp.int32),
            pltpu.VMEM((2, GCH, WROW), jnp.float32),
            pltpu.VMEM((BPW, N_NODES), jnp.float32),
            pltpu.VMEM((LANES, LANES), jnp.float32),
            pltpu.VMEM((BPW,), jnp.float32),
            pltpu.SemaphoreType.DMA,
            pltpu.SemaphoreType.DMA,
        ],
    )(_sc_dot_kernel)
    return run(t2, c2, hp0, hp1, hp2)


# speedup vs baseline: 1.2389x; 1.2389x over previous
import functools

import jax
import jax.numpy as jnp
from jax import lax
from jax.experimental import pallas as pl
from jax.experimental.pallas import tpu as pltpu
from jax.experimental.pallas import tpu_sc as plsc

N_NODES = 64
N_GROUPS = 100000
BATCH = 16384

NC = 2
NS = 16
NW = NC * NS
LANES = 16

BPW = BATCH // NW
GCH = 128
NG = BPW // GCH
WROW = 128

BM_W = 8192
BM_T = 8192

S_TC = 47104
NBLK = 413
N_SC = NBLK * GCH
TAIL = N_GROUPS - S_TC - N_SC
PREP_ITERS = 13


def _tp_w_kernel(wt_ref, out_ref):
    x = wt_ref[...]
    xt = x.T
    pad = jnp.zeros((BM_W, WROW - N_NODES), jnp.float32)
    out_ref[...] = jnp.concatenate([xt, pad], axis=1)


def _tp_t_kernel(tt_ref, out_ref):
    out_ref[...] = tt_ref[...].T


def _sc_prep_kernel(wt_hbm, hp1_hbm, in_v, out_v, isem, osem):
    sid = lax.axis_index("s") * NC + lax.axis_index("c")
    lane_iota = lax.iota(jnp.int32, LANES)

    def fire_in(kb, slot):
        b = jnp.minimum(kb * NW + sid, NBLK - 1)
        col0 = pl.multiple_of(S_TC + b * GCH, GCH)
        return pltpu.async_copy(wt_hbm.at[:, pl.ds(col0, GCH)],
                                in_v.at[slot], isem)

    def fire_out(kb, slot):
        b = jnp.minimum(kb * NW + sid, NBLK - 1)
        row0 = pl.multiple_of(b * GCH, GCH)
        return pltpu.async_copy(out_v.at[slot, :, pl.ds(0, WROW)],
                                hp1_hbm.at[pl.ds(row0, GCH)], osem)

    def transpose_block(slot):
        ib = in_v.at[slot]
        ob = out_v.at[slot]

        @pl.loop(0, GCH, step=LANES)
        def _cols(q0):
            row_ids = q0 + lane_iota
            for jj in range(0, N_NODES, 8):
                vs = [ib[jj + u, pl.ds(q0, LANES)] for u in range(8)]
                for u in range(8):
                    plsc.store_scatter(
                        ob, [row_ids, jnp.full((LANES,), jj + u, jnp.int32)],
                        vs[u])

    in_cp = {0: fire_in(0, 0), 1: fire_in(1, 1)}
    out_cp = {}
    for kb in range(PREP_ITERS):
        slot = kb % 2
        in_cp[kb].wait()
        if kb >= 2:
            out_cp[kb - 2].wait()
        transpose_block(slot)
        out_cp[kb] = fire_out(kb, slot)
        if kb + 2 < PREP_ITERS:
            in_cp[kb + 2] = fire_in(kb + 2, slot)
    out_cp[PREP_ITERS - 2].wait()
    out_cp[PREP_ITERS - 1].wait()


def _tp_tail_kernel(wt_ref, out_ref):
    x = wt_ref[...]
    pad = jnp.zeros((TAIL, WROW - N_NODES), jnp.float32)
    out_ref[...] = jnp.concatenate([x.T, pad], axis=1)


def _sc_dot_kernel(t_hbm, c_hbm, w0_hbm, w1_hbm, w2_hbm, out_hbm, c_v,
                   idx0_v, idx1_v, idx2_v, rows_v, t_v, buf_v, out_v,
                   gsem, tsem):
    wid = lax.axis_index("s") * NC + lax.axis_index("c")
    base = pl.multiple_of(wid * BPW, BPW)

    pltpu.sync_copy(c_hbm.at[wid], c_v)

    t_cp = pltpu.async_copy(t_hbm.at[pl.ds(base, BPW)], t_v, tsem)

    @pl.loop(0, GCH, step=LANES)
    def _split(j):
        for g in range(NG):
            cv = c_v[g, pl.ds(j, LANES)]
            idx0_v[g, pl.ds(j, LANES)] = jnp.where(cv < S_TC, cv, -1)
            idx1_v[g, pl.ds(j, LANES)] = jnp.where(
                (cv >= S_TC) & (cv < S_TC + N_SC), cv - S_TC, -1)
            idx2_v[g, pl.ds(j, LANES)] = jnp.where(
                cv >= S_TC + N_SC, cv - (S_TC + N_SC), -1)

    def fire(g):
        rb = rows_v.at[g % 2]
        cp0 = pltpu.async_copy(
            w0_hbm.at[plsc.Indices(idx0_v.at[g], ignored_value=-1)], rb, gsem)
        cp1 = pltpu.async_copy(
            w1_hbm.at[plsc.Indices(idx1_v.at[g], ignored_value=-1)], rb, gsem)
        cp2 = pltpu.async_copy(
            w2_hbm.at[plsc.Indices(idx2_v.at[g], ignored_value=-1)], rb, gsem)
        return (cp0, cp1, cp2)

    gathers = [fire(0), fire(1)]
    t_cp.wait()

    lane_iota = lax.iota(jnp.int32, LANES)
    nchunk = N_NODES // LANES

    for g in range(NG):
        for cp_h in gathers[g]:
            cp_h.wait()
        rb = rows_v.at[g % 2]

        @pl.loop(0, GCH, step=16)
        def _group(r0):
            row0 = g * GCH + r0
            for r in range(16):
                lrow = r0 + r
                grow = row0 + r
                acc = (rb[lrow, pl.ds(0, LANES)]
                       * t_v[grow, pl.ds(0, LANES)])
                for k in range(1, nchunk):
                    acc = acc + (rb[lrow, pl.ds(k * LANES, LANES)]
                                 * t_v[grow, pl.ds(k * LANES, LANES)])
                plsc.store_scatter(
                    buf_v, [lane_iota, jnp.full((LANES,), r, jnp.int32)], acc)
            tot = buf_v[0, :]
            for l in range(1, 16):
                tot = tot + buf_v[l, :]
            out_v[pl.ds(row0, 16)] = tot

        if g + 2 < NG:
            gathers.append(fire(g + 2))

    pltpu.sync_copy(out_v, out_hbm.at[pl.ds(base, BPW)])


@jax.jit
def kernel(t, c, W):
    c2 = c.reshape(NW, NG, GCH).astype(jnp.int32)
    wt = W.T
    mesh = plsc.VectorSubcoreMesh(core_axis_name="c", subcore_axis_name="s")
    cp = pltpu.CompilerParams(needs_layout_passes=False)

    hp1 = functools.partial(
        pl.kernel,
        mesh=mesh,
        compiler_params=cp,
        out_type=jax.ShapeDtypeStruct((N_SC, WROW), jnp.float32),
        scratch_types=[
            pltpu.VMEM((2, N_NODES, GCH), jnp.float32),
            pltpu.VMEM((2, GCH, WROW + 1), jnp.float32),
            pltpu.SemaphoreType.DMA,
            pltpu.SemaphoreType.DMA,
        ],
    )(_sc_prep_kernel)(wt)

    tc_params = pltpu.CompilerParams(dimension_semantics=("parallel",))
    hp0 = pl.pallas_call(
        _tp_w_kernel,
        out_shape=jax.ShapeDtypeStruct((S_TC, WROW), jnp.float32),
        grid=((S_TC + BM_W - 1) // BM_W,),
        in_specs=[pl.BlockSpec((N_NODES, BM_W), lambda i: (0, i))],
        out_specs=pl.BlockSpec((BM_W, WROW), lambda i: (i, 0)),
        compiler_params=tc_params,
    )(wt)

    wt_tail = jax.lax.slice(wt, (0, S_TC + N_SC), (N_NODES, N_GROUPS))
    hp2 = pl.pallas_call(
        _tp_tail_kernel,
        out_shape=jax.ShapeDtypeStruct((TAIL, WROW), jnp.float32),
    )(wt_tail)

    t2 = pl.pallas_call(
        _tp_t_kernel,
        out_shape=jax.ShapeDtypeStruct((BATCH, N_NODES), jnp.float32),
        grid=(BATCH // BM_T,),
        in_specs=[pl.BlockSpec((N_NODES, BM_T), lambda i: (0, i))],
        out_specs=pl.BlockSpec((BM_T, N_NODES), lambda i: (i, 0)),
        compiler_params=tc_params,
    )(t.T)

    run = functools.partial(
        pl.kernel,
        mesh=mesh,
        compiler_params=cp,
        out_type=jax.ShapeDtypeStruct((BATCH,), jnp.float32),
        scratch_types=[
            pltpu.VMEM((NG, GCH), jnp.int32),
            pltpu.VMEM((NG, GCH), jnp.int32),
            pltpu.VMEM((NG, GCH), jnp.int32),
            pltpu.VMEM((NG, GCH), jnp.int32),
            pltpu.VMEM((2, GCH, WROW), jnp.float32),
            pltpu.VMEM((BPW, N_NODES), jnp.float32),
            pltpu.VMEM((LANES, LANES), jnp.float32),
            pltpu.VMEM((BPW,), jnp.float32),
            pltpu.SemaphoreType.DMA,
            pltpu.SemaphoreType.DMA,
        ],
    )(_sc_dot_kernel)
    return run(t2, c2, hp0, hp1, hp2)


# speedup vs baseline: 1.9182x; 1.5484x over previous
import functools

import jax
import jax.numpy as jnp
from jax import lax
from jax.experimental import pallas as pl
from jax.experimental.pallas import tpu as pltpu
from jax.experimental.pallas import tpu_sc as plsc

N_NODES = 64
N_GROUPS = 100000
BATCH = 16384

NC = 2
NS = 16
NW = NC * NS
LANES = 16

BPW = BATCH // NW
GCH = 128
NG = BPW // GCH
WROW = 128

BM_W = 16384
BM_T = 8192


def _tp_w_kernel(wt_ref, out_ref):
    x = wt_ref[...]
    xt = x.T
    pad = jnp.zeros((BM_W, WROW - N_NODES), jnp.float32)
    out_ref[...] = jnp.concatenate([xt, pad], axis=1)


def _tp_t_kernel(tt_ref, out_ref):
    out_ref[...] = tt_ref[...].T


def _sc_dot_kernel(t_hbm, c_hbm, w_hbm, out_hbm, idx_v, rows_v, t_v, buf_v,
                   out_v, gsem, tsem):
    wid = lax.axis_index("s") * NC + lax.axis_index("c")
    base = pl.multiple_of(wid * BPW, BPW)
    tbase = pl.multiple_of(wid * (BPW // 2), BPW // 2)

    pltpu.sync_copy(c_hbm.at[wid], idx_v)

    t_cp = pltpu.async_copy(t_hbm.at[pl.ds(base, BPW)], t_v, tsem)

    def fire(g):
        return pltpu.async_copy(w_hbm.at[idx_v.at[g]], rows_v.at[g % 2], gsem)

    gathers = [fire(0), fire(1)]
    t_cp.wait()

    lane_iota = lax.iota(jnp.int32, LANES)
    nchunk = N_NODES // LANES

    for g in range(NG):
        gathers[g].wait()
        rb = rows_v.at[g % 2]

        @pl.loop(0, GCH, step=16)
        def _group(r0):
            row0 = g * GCH + r0
            for r in range(16):
                lrow = r0 + r
                grow = row0 + r
                acc = (rb[lrow, pl.ds(0, LANES)]
                       * t_v[grow, pl.ds(0, LANES)])
                for k in range(1, nchunk):
                    acc = acc + (rb[lrow, pl.ds(k * LANES, LANES)]
                                 * t_v[grow, pl.ds(k * LANES, LANES)])
                plsc.store_scatter(
                    buf_v, [lane_iota, jnp.full((LANES,), r, jnp.int32)], acc)
            tot = buf_v[0, :]
            for l in range(1, 16):
                tot = tot + buf_v[l, :]
            out_v[pl.ds(row0, 16)] = tot

        if g + 2 < NG:
            gathers.append(fire(g + 2))

    pltpu.sync_copy(out_v, out_hbm.at[pl.ds(base, BPW)])


@jax.jit
def kernel(t, c, W):
    c2 = c.reshape(NW, NG, GCH).astype(jnp.int32)

    tc_params = pltpu.CompilerParams(dimension_semantics=("parallel",))
    wp = pl.pallas_call(
        _tp_w_kernel,
        out_shape=jax.ShapeDtypeStruct((N_GROUPS, WROW), jnp.float32),
        grid=((N_GROUPS + BM_W - 1) // BM_W,),
        in_specs=[pl.BlockSpec((N_NODES, BM_W), lambda i: (0, i))],
        out_specs=pl.BlockSpec((BM_W, WROW), lambda i: (i, 0)),
        compiler_params=tc_params,
    )(W.T)

    t2 = pl.pallas_call(
        _tp_t_kernel,
        out_shape=jax.ShapeDtypeStruct((BATCH, N_NODES), jnp.float32),
        grid=(BATCH // BM_T,),
        in_specs=[pl.BlockSpec((N_NODES, BM_T), lambda i: (0, i))],
        out_specs=pl.BlockSpec((BM_T, N_NODES), lambda i: (i, 0)),
        compiler_params=tc_params,
    )(t.T)

    mesh = plsc.VectorSubcoreMesh(core_axis_name="c", subcore_axis_name="s")
    cp = pltpu.CompilerParams(needs_layout_passes=False)
    run = functools.partial(
        pl.kernel,
        mesh=mesh,
        compiler_params=cp,
        out_type=jax.ShapeDtypeStruct((BATCH,), jnp.float32),
        scratch_types=[
            pltpu.VMEM((NG, GCH), jnp.int32),
            pltpu.VMEM((2, GCH, WROW), jnp.float32),
            pltpu.VMEM((BPW, N_NODES), jnp.float32),
            pltpu.VMEM((LANES, LANES), jnp.float32),
            pltpu.VMEM((BPW,), jnp.float32),
            pltpu.SemaphoreType.DMA,
            pltpu.SemaphoreType.DMA,
        ],
    )(_sc_dot_kernel)
    return run(t2, c2, wp)
